# Initial kernel scaffold; baseline (speedup 1.0000x reference)
import jax
import jax.numpy as jnp
from jax.experimental import pallas as pl


def kernel(hidden_states, Wg, w1, w2):
    raise NotImplementedError("write your pallas kernel here")


# speedup vs baseline: 2.4682x; 2.4682x over previous
import functools

import jax
import jax.numpy as jnp
from jax.experimental import pallas as pl
from jax.experimental.pallas import tpu as pltpu

E = 8
TOPK = 2
D = 768
F = 768
T = 2048

_NEG = -1e30


def _moe_dense_body(x_ref, wg_ref, w1_ref, w2_ref, out_ref,
                    i1_ref, i2_ref, wa_ref, wb_ref, acc_ref):
    e = pl.program_id(0)

    @pl.when(e == 0)
    def _router():
        x = x_ref[...]
        logits = jnp.dot(x, wg_ref[...], preferred_element_type=jnp.float32)
        idx = jax.lax.broadcasted_iota(jnp.int32, (T, E), 1)
        m1 = jnp.max(logits, axis=1, keepdims=True)
        i1 = jnp.min(jnp.where(logits == m1, idx, E), axis=1, keepdims=True)
        l2 = jnp.where(idx == i1, _NEG, logits)
        m2 = jnp.max(l2, axis=1, keepdims=True)
        i2 = jnp.min(jnp.where(l2 == m2, idx, E), axis=1, keepdims=True)
        wa = jax.nn.sigmoid(m1 - m2)
        i1_ref[...] = i1
        i2_ref[...] = i2
        wa_ref[...] = wa
        wb_ref[...] = 1.0 - wa
        acc_ref[...] = jnp.zeros_like(acc_ref)

    coeff = (wa_ref[...] * (i1_ref[...] == e).astype(jnp.float32)
             + wb_ref[...] * (i2_ref[...] == e).astype(jnp.float32))

    w1e = w1_ref[0].astype(jnp.bfloat16)
    w2e = w2_ref[0].astype(jnp.bfloat16)
    HALF = T // 2
    for h in range(2):
        xb = x_ref[h * HALF:(h + 1) * HALF, :].astype(jnp.bfloat16)
        hh = jax.lax.dot_general(
            xb, w1e, (((1,), (1,)), ((), ())),
            preferred_element_type=jnp.float32)
        g = hh[:, :F]
        u = hh[:, F:]
        act = (g * jax.nn.sigmoid(g) * u).astype(jnp.bfloat16)
        y = jnp.dot(act, w2e, preferred_element_type=jnp.float32)
        acc_ref[h * HALF:(h + 1) * HALF, :] += coeff[h * HALF:(h + 1) * HALF, :] * y

    @pl.when(e == E - 1)
    def _emit():
        out_ref[...] = acc_ref[...]


@jax.jit
def kernel(hidden_states, Wg, w1, w2):
    return pl.pallas_call(
        _moe_dense_body,
        grid=(E,),
        in_specs=[
            pl.BlockSpec((T, D), lambda e: (0, 0)),
            pl.BlockSpec((D, E), lambda e: (0, 0)),
            pl.BlockSpec((1, 2 * F, D), lambda e: (e, 0, 0)),
            pl.BlockSpec((1, F, D), lambda e: (e, 0, 0)),
        ],
        out_specs=pl.BlockSpec((T, D), lambda e: (0, 0)),
        out_shape=jax.ShapeDtypeStruct((T, D), jnp.float32),
        scratch_shapes=[
            pltpu.VMEM((T, 1), jnp.int32),
            pltpu.VMEM((T, 1), jnp.int32),
            pltpu.VMEM((T, 1), jnp.float32),
            pltpu.VMEM((T, 1), jnp.float32),
            pltpu.VMEM((T, D), jnp.float32),
        ],
    )(hidden_states, Wg, w1, w2)
